# Initial kernel scaffold; baseline (speedup 1.0000x reference)
import jax
import jax.numpy as jnp
from jax.experimental import pallas as pl


def kernel(h_p, target_cluster, psi_W, phi_W, phi_b, mask_neg, mask_pos):
    raise NotImplementedError("write your pallas kernel here")


# speedup vs baseline: 1.4751x; 1.4751x over previous
import functools

import jax
import jax.numpy as jnp
from jax import lax
from jax.experimental import pallas as pl
from jax.experimental.pallas import tpu as pltpu
from jax.experimental.pallas import tpu_sc as plsc

HIDDEN = 768
C = 16
W = 512
B = 1024
BM = 128
NB = B // BM
G = NB + C


def _router_body(h_ref, psi_ref, tc_col_ref, tc_row_ref,
                 p_c_ref, rank_ref, sidx_ref, hist_ref):
    logits = lax.dot_general(
        h_ref[...], psi_ref[...], (((1,), (1,)), ((), ())),
        preferred_element_type=jnp.float32)
    m = jnp.max(logits, axis=1, keepdims=True)
    e = jnp.exp(logits - m)
    p_c_ref[...] = e / jnp.sum(e, axis=1, keepdims=True)

    tcc = tc_col_ref[...]
    tcr = tc_row_ref[...]
    bic = lax.broadcasted_iota(jnp.int32, (B, 1), 0)
    bir = lax.broadcasted_iota(jnp.int32, (1, B), 1)

    before = (tcr < tcc) | ((tcr == tcc) & (bir < bic))
    rank = jnp.sum(jnp.where(before, 1, 0), axis=1, keepdims=True)
    rank_ref[...] = rank

    sel = rank == bir
    sidx_ref[...] = jnp.sum(jnp.where(sel, bic, 0), axis=0, keepdims=True)

    cidc = lax.broadcasted_iota(jnp.int32, (C, 1), 0)
    hist_ref[...] = jnp.sum(jnp.where(tcr == cidc, 1, 0), axis=1,
                            keepdims=True)


def _group_body(wb_ref, wc_ref, vld_ref, off_ref, hist_ref,
                x_ref, w_ref, b_ref, mp_ref, mn_ref, o_ref):
    g = pl.program_id(0)
    c = wc_ref[g]
    blk = wb_ref[g]
    start = off_ref[c]
    cnt = hist_ref[c]
    rows = blk * BM + lax.broadcasted_iota(jnp.int32, (BM, 1), 0)
    rmask = (rows >= start) & (rows < start + cnt)

    @pl.when(vld_ref[g] == 1)
    def _():
        x = jnp.where(rmask, x_ref[...], 0.0)
        acc = jnp.dot(x, w_ref[0], preferred_element_type=jnp.float32)
        vals = acc + b_ref[0]
        f = jnp.where(vals > 0, vals, vals * mp_ref[0]) * mn_ref[0]
        m = jnp.max(f, axis=1, keepdims=True)
        e = jnp.exp(f - m)
        sm = e / jnp.sum(e, axis=1, keepdims=True)
        o_ref[...] = jnp.where(rmask, sm, o_ref[...])


def _sc_gather_rows(table, idx, ncols):
    info = plsc.get_sparse_core_info()
    nw = info.num_cores * info.num_subcores
    bpw = B // nw
    mesh = plsc.VectorSubcoreMesh(core_axis_name="c", subcore_axis_name="s")

    @functools.partial(
        pl.kernel, mesh=mesh,
        out_type=jax.ShapeDtypeStruct((B, ncols), jnp.float32),
        scratch_types=[
            pltpu.VMEM((bpw,), jnp.int32),
            pltpu.VMEM((bpw, ncols), jnp.float32),
            pltpu.SemaphoreType.DMA,
        ],
    )
    def k(table_hbm, idx_hbm, out_hbm, idx_v, rows_v, sem):
        wid = lax.axis_index("s") * info.num_cores + lax.axis_index("c")
        base = wid * bpw
        pltpu.sync_copy(idx_hbm.at[pl.ds(base, bpw)], idx_v)
        pltpu.async_copy(table_hbm.at[idx_v], rows_v, sem).wait()
        pltpu.sync_copy(rows_v, out_hbm.at[pl.ds(base, bpw)])

    return k(table, idx)


def kernel(h_p, target_cluster, psi_W, phi_W, phi_b, mask_neg, mask_pos):
    tc = target_cluster.astype(jnp.int32)
    tc_col = tc.reshape(B, 1)
    tc_row = tc.reshape(1, B)

    p_c, rank2, sidx2, hist2 = pl.pallas_call(
        _router_body,
        out_shape=[
            jax.ShapeDtypeStruct((B, C), jnp.float32),
            jax.ShapeDtypeStruct((B, 1), jnp.int32),
            jax.ShapeDtypeStruct((1, B), jnp.int32),
            jax.ShapeDtypeStruct((C, 1), jnp.int32),
        ],
    )(h_p, psi_W, tc_col, tc_row)

    rank = rank2.reshape(B)
    sidx = sidx2.reshape(B)
    hist = hist2.reshape(C)

    off = jnp.concatenate(
        [jnp.zeros((1,), jnp.int32), jnp.cumsum(hist)[:-1].astype(jnp.int32)])
    starts = (jnp.arange(NB, dtype=jnp.int32) * BM)[:, None]
    seg_lo = off[None, :]
    seg_hi = (off + hist)[None, :]
    present = (seg_lo < starts + BM) & (seg_hi > starts) & (hist[None, :] > 0)
    flat = present.reshape(-1)
    pos = jnp.cumsum(flat.astype(jnp.int32)) - 1
    total = pos[-1] + 1
    blk_flat = jnp.repeat(jnp.arange(NB, dtype=jnp.int32), C)
    cl_flat = jnp.tile(jnp.arange(C, dtype=jnp.int32), NB)
    tgt = jnp.where(flat, pos, G)
    wb = jnp.full((G,), NB - 1, jnp.int32).at[tgt].set(blk_flat, mode="drop")
    wc0 = jnp.zeros((G,), jnp.int32).at[tgt].set(cl_flat, mode="drop")
    gi = jnp.arange(G, dtype=jnp.int32)
    wc = jnp.where(gi < total, wc0, jnp.take(wc0, total - 1))
    valid = (gi < total).astype(jnp.int32)

    h_sorted = _sc_gather_rows(h_p, sidx, HIDDEN)

    spec = lambda bs, im: pl.BlockSpec(bs, im)
    grid_spec = pltpu.PrefetchScalarGridSpec(
        num_scalar_prefetch=5,
        grid=(G,),
        in_specs=[
            spec((BM, HIDDEN), lambda g, wb, wc, v, o, h: (wb[g], 0)),
            spec((1, HIDDEN, W), lambda g, wb, wc, v, o, h: (wc[g], 0, 0)),
            spec((1, 1, W), lambda g, wb, wc, v, o, h: (wc[g], 0, 0)),
            spec((1, 1, W), lambda g, wb, wc, v, o, h: (wc[g], 0, 0)),
            spec((1, 1, W), lambda g, wb, wc, v, o, h: (wc[g], 0, 0)),
        ],
        out_specs=spec((BM, W), lambda g, wb, wc, v, o, h: (wb[g], 0)),
    )
    p_w_sorted = pl.pallas_call(
        _group_body,
        grid_spec=grid_spec,
        out_shape=jax.ShapeDtypeStruct((B, W), jnp.float32),
    )(wb, wc, valid, off, hist,
      h_sorted, phi_W, phi_b.reshape(C, 1, W),
      mask_pos.reshape(C, 1, W), mask_neg.reshape(C, 1, W))

    p_w = _sc_gather_rows(p_w_sorted, rank, W)

    return (p_c, p_w)
